# baseline (speedup 1.0000x reference)
import functools

import jax
import jax.numpy as jnp
from jax import lax
from jax.experimental import pallas as pl
from jax.experimental.pallas import tpu as pltpu
from jax.experimental.pallas import tpu_sc as plsc

B = 16384
D = 64
NROW = 1000000
TG = NROW // 8
CHG = 32
CROWS = CHG * 8
NCHK = -(-TG // CHG)
MAXST = TG - CHG
SROWS = 7816

_INFO = plsc.get_sparse_core_info()
NC, NS, L = _INFO.num_cores, _INFO.num_subcores, _INFO.num_lanes
NW = NC * NS
PAIRS = -(-NCHK // NW // 2)
BPW = B // NW
BLK = 128

_mesh = plsc.VectorSubcoreMesh(core_axis_name="c", subcore_axis_name="s")

_DNUMS = lax.GatherDimensionNumbers(
    offset_dims=(), collapsed_slice_dims=(0,), start_index_map=(0,))
_PIB = lax.GatherScatterMode.PROMISE_IN_BOUNDS


def _lane_gather(vec, idx):
    return lax.gather(vec, idx[:, None], _DNUMS, (1,), mode=_PIB)


@functools.partial(
    pl.kernel,
    mesh=_mesh,
    out_type=[jax.ShapeDtypeStruct((SROWS * BLK,), jnp.float32)] * 2,
    compiler_params=pltpu.CompilerParams(needs_layout_passes=False),
    scratch_types=[
        pltpu.VMEM((CHG, 8, D), jnp.float32),
        pltpu.VMEM((CHG, 8, D), jnp.float32),
        pltpu.VMEM((CROWS,), jnp.float32),
        pltpu.VMEM((CROWS,), jnp.float32),
        pltpu.VMEM((2 * D,), jnp.float32),
        pltpu.SemaphoreType.DMA,
        pltpu.SemaphoreType.DMA,
        pltpu.SemaphoreType.DMA,
        pltpu.SemaphoreType.DMA,
    ],
)
def _sc_scores(ut_hbm, it_hbm, w_hbm, su_hbm, si_hbm,
               buf_a, buf_b, sb_a, sb_b, w_v, sem_a, sem_b, sem_oa, sem_ob):
    wid = lax.axis_index("s") * NC + lax.axis_index("c")
    iota = lax.iota(jnp.int32, L)
    pltpu.sync_copy(w_hbm.at[0], w_v)
    ut3 = ut_hbm.reshape(TG, 8, D)
    it3 = it_hbm.reshape(TG, 8, D)

    def start_of(t):
        return jnp.minimum(CHG * wid + (CHG * NW) * t, MAXST)

    def table_pass(tab_ref, s_ref, w_off):
        wv = [w_v[pl.ds(w_off + c * L, L)] for c in range(D // L)]

        def fire(t, buf, sem):
            pltpu.async_copy(tab_ref.at[pl.ds(start_of(t), CHG)], buf, sem)

        def drain(buf, sem):
            pltpu.make_async_copy(tab_ref.at[pl.ds(0, CHG)], buf, sem).wait()

        def fire_out(sbuf, t, sem):
            pltpu.async_copy(sbuf, s_ref.at[pl.ds(8 * start_of(t), CROWS)],
                             sem)

        def drain_out(sbuf, sem):
            pltpu.make_async_copy(s_ref.at[pl.ds(0, CROWS)], sbuf, sem).wait()

        def compute(buf, sbuf):
            def group(g, carry):
                acc = jnp.zeros((L,), jnp.float32)
                for lane in range(L):
                    k = 2 * g + (lane // 8)
                    s = lane % 8
                    v = buf[k, s, pl.ds(0, L)] * wv[0]
                    for c in range(1, D // L):
                        v = v + buf[k, s, pl.ds(c * L, L)] * wv[c]
                    for sh in (8, 4, 2, 1):
                        v = v + _lane_gather(v, jnp.bitwise_xor(iota, sh))
                    acc = jnp.where(iota == lane, v, acc)
                sbuf[pl.ds(g * L, L)] = acc
                return carry
            lax.fori_loop(0, CROWS // L, group, 0)

        fire(0, buf_a, sem_a)
        fire(1, buf_b, sem_b)
        pltpu.async_copy(s_ref.at[pl.ds(0, CROWS)], sb_a, sem_oa)
        pltpu.async_copy(s_ref.at[pl.ds(0, CROWS)], sb_b, sem_ob)

        def pair(p, carry):
            drain(buf_a, sem_a)
            drain_out(sb_a, sem_oa)
            compute(buf_a, sb_a)
            fire_out(sb_a, 2 * p, sem_oa)
            fire(2 * p + 2, buf_a, sem_a)
            drain(buf_b, sem_b)
            drain_out(sb_b, sem_ob)
            compute(buf_b, sb_b)
            fire_out(sb_b, 2 * p + 1, sem_ob)
            fire(2 * p + 3, buf_b, sem_b)
            return carry
        lax.fori_loop(0, PAIRS, pair, 0)

        drain(buf_a, sem_a)
        drain(buf_b, sem_b)
        drain_out(sb_a, sem_oa)
        drain_out(sb_b, sem_ob)

    table_pass(ut3, su_hbm, 0)
    table_pass(it3, si_hbm, D)


@functools.partial(
    pl.kernel,
    mesh=_mesh,
    out_type=jax.ShapeDtypeStruct((B,), jnp.float32),
    compiler_params=pltpu.CompilerParams(needs_layout_passes=False),
    scratch_types=[
        pltpu.VMEM((BPW,), jnp.int32),
        pltpu.VMEM((BPW,), jnp.int32),
        pltpu.VMEM((BPW // BLK, BLK), jnp.int32),
        pltpu.VMEM((BPW // BLK, BLK), jnp.int32),
        pltpu.VMEM((BLK, BLK), jnp.float32),
        pltpu.VMEM((BLK, BLK), jnp.float32),
        pltpu.VMEM((BLK, BLK), jnp.float32),
        pltpu.VMEM((BLK, BLK), jnp.float32),
        pltpu.VMEM((L,), jnp.float32),
        pltpu.VMEM((BPW,), jnp.float32),
        pltpu.SemaphoreType.DMA,
        pltpu.SemaphoreType.DMA,
    ],
)
def _sc_combine(user_hbm, item_hbm, su_hbm, si_hbm, b_hbm, out_hbm,
                uix, iix, ur, ir, buf_ue, buf_ie, buf_uo, buf_io,
                b_v, out_v, sem_e, sem_o):
    wid = lax.axis_index("s") * NC + lax.axis_index("c")
    base = wid * BPW
    iota = lax.iota(jnp.int32, L)
    nblk = BPW // BLK

    pltpu.sync_copy(user_hbm.at[pl.ds(base, BPW)], uix)
    pltpu.sync_copy(item_hbm.at[pl.ds(base, BPW)], iix)

    def rows_body(t, carry):
        k = t // (BLK // L)
        o = (t % (BLK // L)) * L
        ur[k, pl.ds(o, L)] = lax.shift_right_logical(
            uix[pl.ds(k * BLK + o, L)], 7)
        ir[k, pl.ds(o, L)] = lax.shift_right_logical(
            iix[pl.ds(k * BLK + o, L)], 7)
        return carry
    lax.fori_loop(0, nblk * (BLK // L), rows_body, 0)

    def fire(k, buf_u, buf_i, sem):
        pltpu.async_copy(su_hbm.at[ur.at[k]], buf_u, sem)
        pltpu.async_copy(si_hbm.at[ir.at[k]], buf_i, sem)

    def drain(buf_u, buf_i, sem):
        pltpu.make_async_copy(su_hbm.at[pl.ds(0, BLK)], buf_u, sem).wait()
        pltpu.make_async_copy(si_hbm.at[pl.ds(0, BLK)], buf_i, sem).wait()

    fire(0, buf_ue, buf_ie, sem_e)
    fire(1, buf_uo, buf_io, sem_o)

    pltpu.sync_copy(b_hbm, b_v.at[pl.ds(0, 1)])
    bias_bc = _lane_gather(b_v[...], jnp.zeros((L,), jnp.int32))

    def select_block(k, buf_u, buf_i):
        def body(g, carry):
            rix = iota + g * L
            cu = jnp.bitwise_and(uix[pl.ds(k * BLK + g * L, L)],
                                 jnp.int32(127))
            ci = jnp.bitwise_and(iix[pl.ds(k * BLK + g * L, L)],
                                 jnp.int32(127))
            vu = plsc.load_gather(buf_u, [rix, cu])
            vi = plsc.load_gather(buf_i, [rix, ci])
            out_v[pl.ds(k * BLK + g * L, L)] = vu + vi + bias_bc
            return carry
        lax.fori_loop(0, BLK // L, body, 0)

    drain(buf_ue, buf_ie, sem_e)
    select_block(0, buf_ue, buf_ie)
    fire(2, buf_ue, buf_ie, sem_e)
    drain(buf_uo, buf_io, sem_o)
    select_block(1, buf_uo, buf_io)
    fire(3, buf_uo, buf_io, sem_o)
    drain(buf_ue, buf_ie, sem_e)
    select_block(2, buf_ue, buf_ie)
    drain(buf_uo, buf_io, sem_o)
    select_block(3, buf_uo, buf_io)

    pltpu.sync_copy(out_v, out_hbm.at[pl.ds(base, BPW)])


def kernel(user, item, user_table, item_table, lin_w, lin_b):
    s1u, s1i = _sc_scores(user_table, item_table, lin_w)
    su = s1u.reshape(SROWS, BLK)
    si = s1i.reshape(SROWS, BLK)
    out = _sc_combine(user.astype(jnp.int32), item.astype(jnp.int32),
                      su, si, lin_b)
    return out.reshape(B, 1)


# speedup vs baseline: 1.6000x; 1.6000x over previous
import functools

import jax
import jax.numpy as jnp
from jax import lax
from jax.experimental import pallas as pl
from jax.experimental.pallas import tpu as pltpu
from jax.experimental.pallas import tpu_sc as plsc

B = 16384
D = 64
ROWW = 128
_INFO = plsc.get_sparse_core_info()
NC, NS, L = _INFO.num_cores, _INFO.num_subcores, _INFO.num_lanes
NW = NC * NS
BPW = B // NW
HALF = BPW // 2
HGRP = HALF // L

_mesh = plsc.VectorSubcoreMesh(core_axis_name="c", subcore_axis_name="s")

_DNUMS = lax.GatherDimensionNumbers(
    offset_dims=(), collapsed_slice_dims=(0,), start_index_map=(0,))
_PIB = lax.GatherScatterMode.PROMISE_IN_BOUNDS


def _lane_gather(vec, idx):
    return lax.gather(vec, idx[:, None], _DNUMS, (1,), mode=_PIB)


@functools.partial(
    pl.kernel,
    mesh=_mesh,
    out_type=jax.ShapeDtypeStruct((B,), jnp.float32),
    compiler_params=pltpu.CompilerParams(needs_layout_passes=False),
    scratch_types=[
        pltpu.VMEM((BPW,), jnp.int32),
        pltpu.VMEM((BPW,), jnp.int32),
        pltpu.VMEM((HALF, ROWW), jnp.float32),
        pltpu.VMEM((HALF, ROWW), jnp.float32),
        pltpu.VMEM((HALF, ROWW), jnp.float32),
        pltpu.VMEM((HALF * D,), jnp.float32),
        pltpu.VMEM((2 * D,), jnp.float32),
        pltpu.VMEM((L,), jnp.float32),
        pltpu.VMEM((BPW,), jnp.float32),
        pltpu.SemaphoreType.DMA,
        pltpu.SemaphoreType.DMA,
        pltpu.SemaphoreType.DMA,
    ],
)
def _sc_recsys(user_hbm, item_hbm, ut_hbm, it_hbm, w_hbm, b_hbm, out_hbm,
               uix, iix, buf_a, buf_b, buf_c, drain_v, w_v, b_v, out_v,
               sem_a, sem_b, sem_c):
    wid = lax.axis_index("s") * NC + lax.axis_index("c")
    base = wid * BPW
    iota = lax.iota(jnp.int32, L)

    pltpu.sync_copy(user_hbm.at[pl.ds(base, BPW)], uix)
    pltpu.sync_copy(item_hbm.at[pl.ds(base, BPW)], iix)

    def fire_block(idx_ref, table_ref, buf_ref, sem, blk):
        def body(g, carry):
            vec = idx_ref[pl.ds(blk * HALF + g * L, L)]
            for lane in range(L):
                i = jnp.max(jnp.where(iota == lane, vec, jnp.int32(0)))
                pltpu.async_copy(
                    table_ref.at[i], buf_ref.at[g * L + lane, pl.ds(0, D)],
                    sem)
            return carry
        lax.fori_loop(0, HGRP, body, 0)

    def drain(sem):
        pltpu.make_async_copy(out_hbm, drain_v, sem).wait()

    fire_block(uix, ut_hbm, buf_a, sem_a, 0)
    fire_block(uix, ut_hbm, buf_b, sem_b, 1)
    fire_block(iix, it_hbm, buf_c, sem_c, 0)

    pltpu.sync_copy(w_hbm.at[0], w_v)
    pltpu.sync_copy(b_hbm, b_v.at[pl.ds(0, 1)])
    bias_bc = _lane_gather(b_v[...], jnp.zeros((L,), jnp.int32))

    def compute_block(buf_ref, w_off, out_off, first):
        def body(g, carry):
            rix = iota + g * L
            if first:
                acc = bias_bc
            else:
                acc = out_v[pl.ds(out_off + g * L, L)]
            for c in range(D // L):
                wv = w_v[pl.ds(w_off + c * L, L)]
                for jj in range(L):
                    wbc = _lane_gather(wv, jnp.full((L,), jj, jnp.int32))
                    cix = jnp.full((L,), c * L + jj, jnp.int32)
                    vals = plsc.load_gather(buf_ref, [rix, cix])
                    acc = acc + vals * wbc
            out_v[pl.ds(out_off + g * L, L)] = acc
            return carry
        lax.fori_loop(0, HGRP, body, 0)

    drain(sem_a)
    compute_block(buf_a, 0, 0, True)
    fire_block(iix, it_hbm, buf_a, sem_a, 1)
    drain(sem_b)
    compute_block(buf_b, 0, HALF, True)
    drain(sem_c)
    compute_block(buf_c, D, 0, False)
    drain(sem_a)
    compute_block(buf_a, D, HALF, False)

    pltpu.sync_copy(out_v, out_hbm.at[pl.ds(base, BPW)])


def kernel(user, item, user_table, item_table, lin_w, lin_b):
    out = _sc_recsys(user.astype(jnp.int32), item.astype(jnp.int32),
                     user_table, item_table, lin_w, lin_b)
    return out.reshape(B, 1)
